# Initial kernel scaffold; baseline (speedup 1.0000x reference)
import functools

import jax
import jax.numpy as jnp
from jax import lax
from jax.experimental import pallas as pl
from jax.experimental.pallas import tpu as pltpu
from jax.experimental.pallas import tpu_sc as plsc

NC = 2
NS = 16
NW = NC * NS
B = 128


def _sc_aggregate(n_pad, d, k_chunks):
  rows_per = n_pad // NS

  mesh = plsc.VectorSubcoreMesh(core_axis_name="c", subcore_axis_name="s")

  @functools.partial(
      pl.kernel,
      mesh=mesh,
      out_type=[
          jax.ShapeDtypeStruct((NC, n_pad, d), jnp.float32),
          jax.ShapeDtypeStruct((NC, n_pad), jnp.float32),
      ],
      scratch_types=[
          pltpu.VMEM((B,), jnp.int32),
          pltpu.VMEM((B,), jnp.int32),
          pltpu.VMEM((B, d), jnp.float32),
          pltpu.VMEM((B,), jnp.float32),
          pltpu.VMEM_SHARED((n_pad, d), jnp.float32),
          pltpu.VMEM_SHARED((n_pad,), jnp.float32),
          pltpu.SemaphoreType.DMA,
      ],
  )
  def agg(table_hbm, src_hbm, dst_hbm, z2_hbm, z1_hbm, ones_hbm,
          psum_out, cnt_out,
          src_v, dst_v, rows_v, ones_v, accum, cnt_acc, sem):
    c = lax.axis_index("c")
    s = lax.axis_index("s")
    r0 = s * rows_per

    pltpu.sync_copy(z2_hbm.at[pl.ds(r0, rows_per)],
                    accum.at[pl.ds(r0, rows_per)])
    pltpu.sync_copy(z1_hbm.at[pl.ds(r0, rows_per)],
                    cnt_acc.at[pl.ds(r0, rows_per)])
    pltpu.sync_copy(ones_hbm, ones_v)
    plsc.subcore_barrier()

    def body(k, carry):
      pltpu.sync_copy(src_hbm.at[c, s, k], src_v)
      pltpu.sync_copy(dst_hbm.at[c, s, k], dst_v)
      pltpu.async_copy(table_hbm.at[src_v], rows_v, sem).wait()
      pltpu.sync_copy(rows_v, accum.at[dst_v], add=True)
      pltpu.sync_copy(ones_v, cnt_acc.at[dst_v], add=True)
      return carry

    lax.fori_loop(0, k_chunks, body, 0)
    plsc.subcore_barrier()

    pltpu.sync_copy(accum.at[pl.ds(r0, rows_per)],
                    psum_out.at[c, pl.ds(r0, rows_per)])
    pltpu.sync_copy(cnt_acc.at[pl.ds(r0, rows_per)],
                    cnt_out.at[c, pl.ds(r0, rows_per)])

  return agg


def _tc_layer(n, n_pad, d, relu):
  r = 2000
  dot = functools.partial(
      lax.dot_general,
      dimension_numbers=(((1,), (1,)), ((), ())),
      preferred_element_type=jnp.float32,
  )

  def body(x_ref, p_ref, c_ref, wl_ref, wr_ref, b_ref, o_ref):
    cnt = jnp.maximum(c_ref[0] + c_ref[1], 1.0)
    mean = (p_ref[0] + p_ref[1]) / cnt
    h = dot(mean, wl_ref[...]) + dot(x_ref[...], wr_ref[...]) + b_ref[...]
    if relu:
      h = jnp.maximum(h, 0.0)
    o_ref[...] = h

  return pl.pallas_call(
      body,
      grid=(n // r,),
      in_specs=[
          pl.BlockSpec((r, d), lambda i: (i, 0)),
          pl.BlockSpec((NC, r, d), lambda i: (0, i, 0)),
          pl.BlockSpec((NC, r, 1), lambda i: (0, i, 0)),
          pl.BlockSpec((d, d), lambda i: (0, 0)),
          pl.BlockSpec((d, d), lambda i: (0, 0)),
          pl.BlockSpec((1, d), lambda i: (0, 0)),
      ],
      out_specs=pl.BlockSpec((r, d), lambda i: (i, 0)),
      out_shape=jax.ShapeDtypeStruct((n, d), jnp.float32),
  )


def kernel(x, edge_index, W1_l, b1, W1_r, W2_l, b2, W2_r):
  n, d = x.shape
  e = edge_index.shape[1]

  k_chunks = -(-e // (NW * B))
  e_pad = NW * B * k_chunks
  n_pad = -(-(n + 1) // (NS * 8)) * (NS * 8)

  src = edge_index[0]
  dst = edge_index[1]
  pad = e_pad - e
  src4 = jnp.concatenate([src, jnp.zeros((pad,), jnp.int32)]).reshape(
      NC, NS, k_chunks, B)
  dst4 = jnp.concatenate([dst, jnp.full((pad,), n, jnp.int32)]).reshape(
      NC, NS, k_chunks, B)

  z2 = jnp.zeros((n_pad, d), jnp.float32)
  z1 = jnp.zeros((n_pad,), jnp.float32)
  ones = jnp.ones((B,), jnp.float32)

  agg = _sc_aggregate(n_pad, d, k_chunks)
  l1 = _tc_layer(n, n_pad, d, relu=True)
  l2 = _tc_layer(n, n_pad, d, relu=False)

  b1r = b1.reshape(1, d)
  b2r = b2.reshape(1, d)

  p1, c1 = agg(x, src4, dst4, z2, z1, ones)
  c1r = c1.reshape(NC, n_pad, 1)
  h = l1(x, p1, c1r, W1_l, W1_r, b1r)
  p2, _ = agg(h, src4, dst4, z2, z1, ones)
  out = l2(h, p2, c1r, W2_l, W2_r, b2r)
  return out


# speedup vs baseline: 3.7704x; 3.7704x over previous
import functools

import jax
import jax.numpy as jnp
from jax import lax
from jax.experimental import pallas as pl
from jax.experimental.pallas import tpu as pltpu
from jax.experimental.pallas import tpu_sc as plsc

NC = 2
NS = 16
NW = NC * NS
B = 128


def _sc_aggregate(n_pad, d, k_chunks):
  rows_per = n_pad // NS

  mesh = plsc.VectorSubcoreMesh(core_axis_name="c", subcore_axis_name="s")

  @functools.partial(
      pl.kernel,
      mesh=mesh,
      out_type=[
          jax.ShapeDtypeStruct((NC, n_pad, d), jnp.float32),
          jax.ShapeDtypeStruct((NC * n_pad,), jnp.float32),
      ],
      scratch_types=[
          pltpu.VMEM((B,), jnp.int32),
          pltpu.VMEM((B,), jnp.int32),
          pltpu.VMEM((B, d), jnp.float32),
          pltpu.VMEM((B,), jnp.float32),
          pltpu.VMEM((rows_per,), jnp.float32),
          pltpu.VMEM_SHARED((n_pad, d), jnp.float32),
          pltpu.VMEM_SHARED((n_pad,), jnp.float32),
          pltpu.SemaphoreType.DMA,
      ],
  )
  def agg(table_hbm, src_hbm, dst_hbm, z2_hbm, z1_hbm, ones_hbm,
          psum_out, cnt_out,
          src_v, dst_v, rows_v, ones_v, cnt_v, accum, cnt_acc, sem):
    c = lax.axis_index("c")
    s = lax.axis_index("s")
    r0 = s * rows_per

    pltpu.sync_copy(z2_hbm.at[pl.ds(r0, rows_per)],
                    accum.at[pl.ds(r0, rows_per)])
    pltpu.sync_copy(z1_hbm.at[pl.ds(r0, rows_per)], cnt_v)
    pltpu.sync_copy(cnt_v, cnt_acc.at[pl.ds(r0, rows_per)])
    pltpu.sync_copy(ones_hbm, ones_v)
    plsc.subcore_barrier()

    def body(k, carry):
      pltpu.sync_copy(src_hbm.at[c, s, k], src_v)
      pltpu.sync_copy(dst_hbm.at[c, s, k], dst_v)
      pltpu.async_copy(table_hbm.at[src_v], rows_v, sem).wait()
      pltpu.sync_copy(rows_v, accum.at[dst_v], add=True)
      pltpu.sync_copy(ones_v, cnt_acc.at[dst_v], add=True)
      return carry

    lax.fori_loop(0, k_chunks, body, 0)
    plsc.subcore_barrier()

    pltpu.sync_copy(accum.at[pl.ds(r0, rows_per)],
                    psum_out.at[c, pl.ds(r0, rows_per)])
    pltpu.sync_copy(cnt_acc.at[pl.ds(r0, rows_per)], cnt_v)
    pltpu.sync_copy(cnt_v, cnt_out.at[pl.ds(c * n_pad + r0, rows_per)])

  return agg


def _tc_layer(n, n_pad, d, relu):
  r = 2000
  dot = functools.partial(
      lax.dot_general,
      dimension_numbers=(((1,), (1,)), ((), ())),
      preferred_element_type=jnp.float32,
  )

  def body(x_ref, p_ref, c_ref, wl_ref, wr_ref, b_ref, o_ref):
    cnt = jnp.maximum(c_ref[0] + c_ref[1], 1.0)
    mean = (p_ref[0] + p_ref[1]) / cnt
    h = dot(mean, wl_ref[...]) + dot(x_ref[...], wr_ref[...]) + b_ref[...]
    if relu:
      h = jnp.maximum(h, 0.0)
    o_ref[...] = h

  return pl.pallas_call(
      body,
      grid=(n // r,),
      in_specs=[
          pl.BlockSpec((r, d), lambda i: (i, 0)),
          pl.BlockSpec((NC, r, d), lambda i: (0, i, 0)),
          pl.BlockSpec((NC, r, 1), lambda i: (0, i, 0)),
          pl.BlockSpec((d, d), lambda i: (0, 0)),
          pl.BlockSpec((d, d), lambda i: (0, 0)),
          pl.BlockSpec((1, d), lambda i: (0, 0)),
      ],
      out_specs=pl.BlockSpec((r, d), lambda i: (i, 0)),
      out_shape=jax.ShapeDtypeStruct((n, d), jnp.float32),
  )


def kernel(x, edge_index, W1_l, b1, W1_r, W2_l, b2, W2_r):
  n, d = x.shape
  e = edge_index.shape[1]

  k_chunks = -(-e // (NW * B))
  e_pad = NW * B * k_chunks
  n_pad = -(-(n + 1) // (NS * 8)) * (NS * 8)

  src = edge_index[0]
  dst = edge_index[1]
  pad = e_pad - e
  src4 = jnp.concatenate([src, jnp.zeros((pad,), jnp.int32)]).reshape(
      NC, NS, k_chunks, B)
  dst4 = jnp.concatenate([dst, jnp.full((pad,), n, jnp.int32)]).reshape(
      NC, NS, k_chunks, B)

  z2 = jnp.zeros((n_pad, d), jnp.float32)
  z1 = jnp.zeros((n_pad,), jnp.float32)
  ones = jnp.ones((B,), jnp.float32)

  agg = _sc_aggregate(n_pad, d, k_chunks)
  l1 = _tc_layer(n, n_pad, d, relu=True)
  l2 = _tc_layer(n, n_pad, d, relu=False)

  b1r = b1.reshape(1, d)
  b2r = b2.reshape(1, d)

  p1, c1 = agg(x, src4, dst4, z2, z1, ones)
  c1r = c1.reshape(NC, n_pad, 1)
  h = l1(x, p1, c1r, W1_l, W1_r, b1r)
  p2, _ = agg(h, src4, dst4, z2, z1, ones)
  out = l2(h, p2, c1r, W2_l, W2_r, b2r)
  return out
